# Initial kernel scaffold; baseline (speedup 1.0000x reference)
import jax
import jax.numpy as jnp
from jax.experimental import pallas as pl


def kernel(params, idx, targets):
    raise NotImplementedError("write your pallas kernel here")


# speedup vs baseline: 1.2930x; 1.2930x over previous
import functools

import jax
import jax.numpy as jnp
import numpy as np
from jax import lax
from jax.experimental import pallas as pl
from jax.experimental.pallas import tpu as pltpu
from jax.experimental.pallas import tpu_sc as plsc

V = 32000
D = 768
NH = 12
HD = D // NH
E = 8
HID = int(2 * D / 3) * 4
T = 2048
THETA = 10000.0

_freqs = 1.0 / THETA ** (np.arange(0, HD, 2)[: HD // 2].astype(np.float32) / HD)
_ang = np.outer(np.arange(T * 2).astype(np.float32), _freqs)
_COS = jnp.asarray(np.cos(_ang), dtype=jnp.float32)
_SIN = jnp.asarray(np.sin(_ang), dtype=jnp.float32)

RB = 256
VB = 1280
NV = V // VB
BM = 128
PAD = 2 * T + E * BM
NBLK = PAD // BM


def _norm(x, w):
    xf = x.astype(jnp.float32)
    out = xf * lax.rsqrt(jnp.mean(xf * xf, axis=-1, keepdims=True) + 1e-5)
    return out.astype(x.dtype) * w


def _rotary(x):
    t = x.shape[1]
    xr = x.astype(jnp.float32).reshape(x.shape[0], x.shape[1], x.shape[2], -1, 2)
    x0 = xr[..., 0]
    x1 = xr[..., 1]
    c = _COS[:t].reshape(1, t, 1, -1)
    s = _SIN[:t].reshape(1, t, 1, -1)
    o0 = x0 * c - x1 * s
    o1 = x0 * s + x1 * c
    return jnp.stack([o0, o1], axis=-1).reshape(x.shape).astype(x.dtype)


def _attention(h, lp):
    b, t, c = h.shape
    xq = (h @ lp['wq']).reshape(b, t, NH, HD)
    xk = (h @ lp['wk']).reshape(b, t, NH, HD)
    xv = (h @ lp['wv']).reshape(b, t, NH, HD)
    xq = _rotary(xq)
    xk = _rotary(xk)
    q = xq.transpose(0, 2, 1, 3)
    k = xk.transpose(0, 2, 1, 3)
    v = xv.transpose(0, 2, 1, 3)
    scores = (q @ k.transpose(0, 1, 3, 2)) / jnp.sqrt(jnp.float32(HD))
    mask = jnp.tril(jnp.ones((t, t), dtype=bool))
    scores = jnp.where(mask[None, None], scores, jnp.float32(-1e9))
    p = jax.nn.softmax(scores, axis=-1)
    y = (p @ v).transpose(0, 2, 1, 3).reshape(b, t, c)
    return y @ lp['wo']


def _gather_rows(table, idx):
    info = plsc.get_sparse_core_info()
    nc, ns = info.num_cores, info.num_subcores
    bpw = T // (nc * ns)
    mesh = plsc.VectorSubcoreMesh(core_axis_name="c", subcore_axis_name="s")

    @functools.partial(
        pl.kernel,
        mesh=mesh,
        out_type=jax.ShapeDtypeStruct((T, D), jnp.float32),
        scratch_types=[
            pltpu.VMEM((bpw,), jnp.int32),
            pltpu.VMEM((bpw, D), jnp.float32),
            pltpu.SemaphoreType.DMA,
        ],
    )
    def k(table_hbm, idx_hbm, out_hbm, idx_v, rows_v, sem):
        wid = lax.axis_index("s") * nc + lax.axis_index("c")
        base = wid * bpw
        pltpu.sync_copy(idx_hbm.at[pl.ds(base, bpw)], idx_v)
        pltpu.async_copy(table_hbm.at[idx_v], rows_v, sem).wait()
        pltpu.sync_copy(rows_v, out_hbm.at[pl.ds(base, bpw)])

    return k(table, idx)


def _embed(table, idx):
    return _gather_rows(table, idx)


def _dispatch(xn, slk0, slk1):
    info = plsc.get_sparse_core_info()
    nc, ns = info.num_cores, info.num_subcores
    bpw = T // (nc * ns)
    mesh = plsc.VectorSubcoreMesh(core_axis_name="c", subcore_axis_name="s")

    @functools.partial(
        pl.kernel,
        mesh=mesh,
        out_type=jax.ShapeDtypeStruct((PAD, D), jnp.float32),
        scratch_types=[
            pltpu.VMEM((bpw,), jnp.int32),
            pltpu.VMEM((bpw,), jnp.int32),
            pltpu.VMEM((bpw, D), jnp.float32),
            pltpu.SemaphoreType.DMA,
            pltpu.SemaphoreType.DMA,
        ],
    )
    def k(xn_hbm, s0_hbm, s1_hbm, out_hbm, i0_v, i1_v, rows_v, sem0, sem1):
        wid = lax.axis_index("s") * nc + lax.axis_index("c")
        base = wid * bpw
        pltpu.sync_copy(xn_hbm.at[pl.ds(base, bpw)], rows_v)
        pltpu.sync_copy(s0_hbm.at[pl.ds(base, bpw)], i0_v)
        pltpu.sync_copy(s1_hbm.at[pl.ds(base, bpw)], i1_v)
        c0 = pltpu.async_copy(rows_v, out_hbm.at[i0_v], sem0)
        c1 = pltpu.async_copy(rows_v, out_hbm.at[i1_v], sem1)
        c0.wait()
        c1.wait()

    return k(xn, slk0, slk1)


def _block_csum(oh):
    tril = (lax.broadcasted_iota(jnp.int32, (128, 128), 0)
            >= lax.broadcasted_iota(jnp.int32, (128, 128), 1)).astype(jnp.float32)
    outs = []
    tot = jnp.zeros((1, E), jnp.float32)
    for c in range(T // 128):
        ch = oh[c * 128:(c + 1) * 128]
        w = jnp.dot(tril, ch, preferred_element_type=jnp.float32)
        outs.append(w + tot)
        tot = tot + jnp.sum(ch, axis=0, keepdims=True)
    return jnp.concatenate(outs, axis=0), tot


def _meta_body(sel_ref, slot_ref, be_ref, tp_ref):
    sel = sel_ref[...]
    s0 = sel[:, 0:1]
    s1 = sel[:, 1:2]
    lanes = lax.broadcasted_iota(jnp.int32, (T, E), 1)
    oh0 = (lanes == s0).astype(jnp.float32)
    oh1 = (lanes == s1).astype(jnp.float32)
    csum0, tot0 = _block_csum(oh0)
    csum1, tot1 = _block_csum(oh1)
    csum1 = csum1 + tot0
    csum = jnp.concatenate([csum0, csum1], axis=0)
    counts = tot0 + tot1
    pc = jnp.ceil(counts * (1.0 / BM)) * BM
    em1 = lax.broadcasted_iota(jnp.int32, (E, E), 0)
    em2 = lax.broadcasted_iota(jnp.int32, (E, E), 1)
    lower = (em1 < em2).astype(jnp.float32)
    off = jnp.dot(pc, lower, preferred_element_type=jnp.float32)
    rank0 = jnp.sum(oh0 * csum[:T], axis=1, keepdims=True) - 1.0
    rank1 = jnp.sum(oh1 * csum[T:], axis=1, keepdims=True) - 1.0
    off0 = jnp.sum(oh0 * off, axis=1, keepdims=True)
    off1 = jnp.sum(oh1 * off, axis=1, keepdims=True)
    slot_ref[...] = jnp.concatenate(
        [off0 + rank0, off1 + rank1], axis=1).astype(jnp.int32)
    ends = off + pc
    bio = (lax.broadcasted_iota(jnp.int32, (NBLK, E), 0) * BM).astype(jnp.float32)
    bef = jnp.sum((bio >= ends).astype(jnp.int32), axis=1, keepdims=True)
    be_ref[...] = jnp.minimum(bef, E - 1)
    tp_ref[...] = jnp.sum(pc).astype(jnp.int32).reshape(1, 1)


def _meta(sel2):
    return pl.pallas_call(
        _meta_body,
        grid=(1,),
        in_specs=[pl.BlockSpec((T, 2), lambda i: (0, 0))],
        out_specs=[
            pl.BlockSpec((T, 2), lambda i: (0, 0)),
            pl.BlockSpec((NBLK, 1), lambda i: (0, 0)),
            pl.BlockSpec((1, 1), lambda i: (0, 0)),
        ],
        out_shape=[
            jax.ShapeDtypeStruct((T, 2), jnp.int32),
            jax.ShapeDtypeStruct((NBLK, 1), jnp.int32),
            jax.ShapeDtypeStruct((1, 1), jnp.int32),
        ],
    )(sel2)


def _egemm_body(be_ref, tp_ref, xg_ref, w1_ref, w3_ref, w2_ref, yg_ref):
    i = pl.program_id(0)

    @pl.when(i * BM < tp_ref[0])
    def _():
        xb = xg_ref[...]
        h1 = jnp.dot(xb, w1_ref[0], preferred_element_type=jnp.float32)
        h1 = h1 * (1.0 / (1.0 + jnp.exp(-h1)))
        h3 = jnp.dot(xb, w3_ref[0], preferred_element_type=jnp.float32)
        yg_ref[...] = jnp.dot(h1 * h3, w2_ref[0],
                              preferred_element_type=jnp.float32)


def _egemm(be, tp, xg, w1, w3, w2):
    grid_spec = pltpu.PrefetchScalarGridSpec(
        num_scalar_prefetch=2,
        grid=(NBLK,),
        in_specs=[
            pl.BlockSpec((BM, D), lambda i, be, tp: (i, 0)),
            pl.BlockSpec((1, D, HID), lambda i, be, tp: (be[i], 0, 0)),
            pl.BlockSpec((1, D, HID), lambda i, be, tp: (be[i], 0, 0)),
            pl.BlockSpec((1, HID, D), lambda i, be, tp: (be[i], 0, 0)),
        ],
        out_specs=pl.BlockSpec((BM, D), lambda i, be, tp: (i, 0)),
    )
    return pl.pallas_call(
        _egemm_body,
        grid_spec=grid_spec,
        out_shape=jax.ShapeDtypeStruct((PAD, D), jnp.float32),
    )(be, tp, xg, w1, w3, w2)


def _combine_body(x_ref, w_ref, sel_ref, y0_ref, y1_ref, o_ref):
    w = w_ref[...]
    sel = sel_ref[...]
    lo = sel[:, 0:1] < sel[:, 1:2]
    wlo = jnp.where(lo, w[:, 0:1], w[:, 1:2])
    whi = jnp.where(lo, w[:, 1:2], w[:, 0:1])
    y0 = y0_ref[...]
    y1 = y1_ref[...]
    ylo = jnp.where(lo, y0, y1)
    yhi = jnp.where(lo, y1, y0)
    o_ref[...] = x_ref[...] + (wlo * ylo + whi * yhi)


def _combine(x, wt2, sel2, y0, y1):
    n = T // RB
    return pl.pallas_call(
        _combine_body,
        grid=(n,),
        in_specs=[
            pl.BlockSpec((RB, D), lambda i: (i, 0)),
            pl.BlockSpec((RB, 2), lambda i: (i, 0)),
            pl.BlockSpec((RB, 2), lambda i: (i, 0)),
            pl.BlockSpec((RB, D), lambda i: (i, 0)),
            pl.BlockSpec((RB, D), lambda i: (i, 0)),
        ],
        out_specs=pl.BlockSpec((RB, D), lambda i: (i, 0)),
        out_shape=jax.ShapeDtypeStruct((T, D), jnp.float32),
    )(x, wt2, sel2, y0, y1)


def _moe_routed(x, xf, sel2, wt2, w1, w3, w2):
    slot2, be, tp = _meta(sel2)
    xg = _dispatch(xf, slot2[:, 0], slot2[:, 1])
    yg = _egemm(be.reshape(NBLK), tp.reshape(1), xg, w1, w3, w2)
    y0 = _gather_rows(yg, slot2[:, 0])
    y1 = _gather_rows(yg, slot2[:, 1])
    return _combine(x, wt2, sel2, y0, y1)


def _rms_body(x_ref, ln_ref, o_ref):
    x = x_ref[...]
    ms = jnp.mean(x * x, axis=-1, keepdims=True)
    o_ref[...] = x * lax.rsqrt(ms + 1e-5) * ln_ref[...]


def _rmsnorm(x, ln):
    n = T // RB
    return pl.pallas_call(
        _rms_body,
        grid=(n,),
        in_specs=[
            pl.BlockSpec((RB, D), lambda i: (i, 0)),
            pl.BlockSpec((1, D), lambda i: (0, 0)),
        ],
        out_specs=pl.BlockSpec((RB, D), lambda i: (i, 0)),
        out_shape=jax.ShapeDtypeStruct((T, D), jnp.float32),
    )(x, ln.reshape(1, D))


def _head_body(xn_ref, wt_ref, tgt_ref, logits_ref, loss_ref,
               m_ref, s_ref, tl_ref):
    v = pl.program_id(0)
    lg = jnp.dot(xn_ref[...], wt_ref[...], preferred_element_type=jnp.float32)
    logits_ref[...] = lg
    col = lax.broadcasted_iota(jnp.int32, (T, VB), 1) + v * VB
    hit = col == tgt_ref[...]
    tl_part = jnp.sum(jnp.where(hit, lg, 0.0), axis=1, keepdims=True)
    rm = jnp.max(lg, axis=1, keepdims=True)

    @pl.when(v == 0)
    def _():
        m_ref[...] = jnp.full((T, 1), -1e30, jnp.float32)
        s_ref[...] = jnp.zeros((T, 1), jnp.float32)
        tl_ref[...] = jnp.zeros((T, 1), jnp.float32)

    m_old = m_ref[...]
    m_new = jnp.maximum(m_old, rm)
    s_ref[...] = s_ref[...] * jnp.exp(m_old - m_new) + jnp.sum(
        jnp.exp(lg - m_new), axis=1, keepdims=True)
    m_ref[...] = m_new
    tl_ref[...] += tl_part

    @pl.when(v == NV - 1)
    def _():
        loss_ref[...] = jnp.mean(
            m_ref[...] + jnp.log(s_ref[...]) - tl_ref[...]).reshape(1, 1)


def _head(xn, wteT, tgt):
    return pl.pallas_call(
        _head_body,
        grid=(NV,),
        in_specs=[
            pl.BlockSpec((T, D), lambda v: (0, 0)),
            pl.BlockSpec((D, VB), lambda v: (0, v)),
            pl.BlockSpec((T, 1), lambda v: (0, 0)),
        ],
        out_specs=[
            pl.BlockSpec((T, VB), lambda v: (0, v)),
            pl.BlockSpec((1, 1), lambda v: (0, 0)),
        ],
        out_shape=[
            jax.ShapeDtypeStruct((T, V), jnp.float32),
            jax.ShapeDtypeStruct((1, 1), jnp.float32),
        ],
        scratch_shapes=[
            pltpu.VMEM((T, 1), jnp.float32),
            pltpu.VMEM((T, 1), jnp.float32),
            pltpu.VMEM((T, 1), jnp.float32),
        ],
    )(xn, wteT, tgt)


def _moe_dense_jax(h, lp):
    b, t, c = h.shape
    xf = h.reshape(-1, c)
    gl = xf @ lp['gate']
    w, sel = jax.lax.top_k(gl, 2)
    w = jax.nn.softmax(w.astype(jnp.float32), axis=-1).astype(xf.dtype)
    out = jnp.zeros_like(xf)
    for j in range(E):
        mw = jnp.sum(w * (sel == j).astype(w.dtype), axis=-1)
        eo = (jax.nn.silu(xf @ lp['w1'][j]) * (xf @ lp['w3'][j])) @ lp['w2'][j]
        out = out + mw[:, None] * eo
    return out.reshape(b, t, c)


def kernel(params, idx, targets):
    wte = params['wte']
    x = _embed(wte, idx.reshape(T)).reshape(1, T, D)
    nl = len(params['layers'])
    for li, lp in enumerate(params['layers']):
        x = x + _attention(_norm(x, lp['ln1']), lp)
        xn2 = _norm(x, lp['ln2'])
        if li < nl - 1:
            x = x + _moe_dense_jax(xn2, lp)
        else:
            xf = xn2.reshape(-1, D)
            gl = xf @ lp['gate']
            w, sel = jax.lax.top_k(gl, 2)
            w = jax.nn.softmax(w.astype(jnp.float32), axis=-1).astype(xf.dtype)
            x = _moe_routed(x.reshape(T, D), xf, sel, w,
                            lp['w1'], lp['w3'], lp['w2']).reshape(1, T, D)
    xn = _rmsnorm(x.reshape(T, D), params['ln_f'])
    logits, loss = _head(xn, wte.T, targets.reshape(T, 1))
    return logits.reshape(1, T, V), loss.reshape(())


# speedup vs baseline: 1.3603x; 1.0520x over previous
import functools

import jax
import jax.numpy as jnp
import numpy as np
from jax import lax
from jax.experimental import pallas as pl
from jax.experimental.pallas import tpu as pltpu
from jax.experimental.pallas import tpu_sc as plsc

V = 32000
D = 768
NH = 12
HD = D // NH
E = 8
HID = int(2 * D / 3) * 4
T = 2048
THETA = 10000.0

_freqs = 1.0 / THETA ** (np.arange(0, HD, 2)[: HD // 2].astype(np.float32) / HD)
_ang = np.outer(np.arange(T * 2).astype(np.float32), _freqs)
_COS = jnp.asarray(np.cos(_ang), dtype=jnp.float32)
_SIN = jnp.asarray(np.sin(_ang), dtype=jnp.float32)

RB = 256
VB = 1280
NV = V // VB
BM = 128
PAD = 2 * T + E * BM
NBLK = PAD // BM


def _norm(x, w):
    xf = x.astype(jnp.float32)
    out = xf * lax.rsqrt(jnp.mean(xf * xf, axis=-1, keepdims=True) + 1e-5)
    return out.astype(x.dtype) * w


def _rotary(x):
    t = x.shape[1]
    xr = x.astype(jnp.float32).reshape(x.shape[0], x.shape[1], x.shape[2], -1, 2)
    x0 = xr[..., 0]
    x1 = xr[..., 1]
    c = _COS[:t].reshape(1, t, 1, -1)
    s = _SIN[:t].reshape(1, t, 1, -1)
    o0 = x0 * c - x1 * s
    o1 = x0 * s + x1 * c
    return jnp.stack([o0, o1], axis=-1).reshape(x.shape).astype(x.dtype)


def _attention(h, lp):
    b, t, c = h.shape
    xq = (h @ lp['wq']).reshape(b, t, NH, HD)
    xk = (h @ lp['wk']).reshape(b, t, NH, HD)
    xv = (h @ lp['wv']).reshape(b, t, NH, HD)
    xq = _rotary(xq)
    xk = _rotary(xk)
    q = xq.transpose(0, 2, 1, 3)
    k = xk.transpose(0, 2, 1, 3)
    v = xv.transpose(0, 2, 1, 3)
    scores = (q @ k.transpose(0, 1, 3, 2)) / jnp.sqrt(jnp.float32(HD))
    mask = jnp.tril(jnp.ones((t, t), dtype=bool))
    scores = jnp.where(mask[None, None], scores, jnp.float32(-1e9))
    p = jax.nn.softmax(scores, axis=-1)
    y = (p @ v).transpose(0, 2, 1, 3).reshape(b, t, c)
    return y @ lp['wo']


def _gather_rows(table, idx):
    info = plsc.get_sparse_core_info()
    nc, ns = info.num_cores, info.num_subcores
    bpw = T // (nc * ns)
    mesh = plsc.VectorSubcoreMesh(core_axis_name="c", subcore_axis_name="s")

    @functools.partial(
        pl.kernel,
        mesh=mesh,
        out_type=jax.ShapeDtypeStruct((T, D), jnp.float32),
        scratch_types=[
            pltpu.VMEM((bpw,), jnp.int32),
            pltpu.VMEM((bpw, D), jnp.float32),
            pltpu.SemaphoreType.DMA,
        ],
    )
    def k(table_hbm, idx_hbm, out_hbm, idx_v, rows_v, sem):
        wid = lax.axis_index("s") * nc + lax.axis_index("c")
        base = wid * bpw
        pltpu.sync_copy(idx_hbm.at[pl.ds(base, bpw)], idx_v)
        pltpu.async_copy(table_hbm.at[idx_v], rows_v, sem).wait()
        pltpu.sync_copy(rows_v, out_hbm.at[pl.ds(base, bpw)])

    return k(table, idx)


def _embed(table, idx):
    return _gather_rows(table, idx)


def _dispatch(xn, slk0, slk1):
    info = plsc.get_sparse_core_info()
    nc, ns = info.num_cores, info.num_subcores
    bpw = T // (nc * ns)
    mesh = plsc.VectorSubcoreMesh(core_axis_name="c", subcore_axis_name="s")

    @functools.partial(
        pl.kernel,
        mesh=mesh,
        out_type=jax.ShapeDtypeStruct((PAD, D), jnp.float32),
        scratch_types=[
            pltpu.VMEM((bpw,), jnp.int32),
            pltpu.VMEM((bpw,), jnp.int32),
            pltpu.VMEM((bpw, D), jnp.float32),
            pltpu.SemaphoreType.DMA,
            pltpu.SemaphoreType.DMA,
        ],
    )
    def k(xn_hbm, s0_hbm, s1_hbm, out_hbm, i0_v, i1_v, rows_v, sem0, sem1):
        wid = lax.axis_index("s") * nc + lax.axis_index("c")
        base = wid * bpw
        pltpu.sync_copy(xn_hbm.at[pl.ds(base, bpw)], rows_v)
        pltpu.sync_copy(s0_hbm.at[pl.ds(base, bpw)], i0_v)
        pltpu.sync_copy(s1_hbm.at[pl.ds(base, bpw)], i1_v)
        c0 = pltpu.async_copy(rows_v, out_hbm.at[i0_v], sem0)
        c1 = pltpu.async_copy(rows_v, out_hbm.at[i1_v], sem1)
        c0.wait()
        c1.wait()

    return k(xn, slk0, slk1)


def _block_csum(oh):
    tril = (lax.broadcasted_iota(jnp.int32, (128, 128), 0)
            >= lax.broadcasted_iota(jnp.int32, (128, 128), 1)).astype(jnp.float32)
    outs = []
    tot = jnp.zeros((1, E), jnp.float32)
    for c in range(T // 128):
        ch = oh[c * 128:(c + 1) * 128]
        w = jnp.dot(tril, ch, preferred_element_type=jnp.float32)
        outs.append(w + tot)
        tot = tot + jnp.sum(ch, axis=0, keepdims=True)
    return jnp.concatenate(outs, axis=0), tot


def _meta_body(sel_ref, slot_ref, be_ref, tp_ref):
    sel = sel_ref[...]
    s0 = sel[:, 0:1]
    s1 = sel[:, 1:2]
    lanes = lax.broadcasted_iota(jnp.int32, (T, E), 1)
    oh0 = (lanes == s0).astype(jnp.float32)
    oh1 = (lanes == s1).astype(jnp.float32)
    csum0, tot0 = _block_csum(oh0)
    csum1, tot1 = _block_csum(oh1)
    csum1 = csum1 + tot0
    csum = jnp.concatenate([csum0, csum1], axis=0)
    counts = tot0 + tot1
    pc = jnp.ceil(counts * (1.0 / BM)) * BM
    em1 = lax.broadcasted_iota(jnp.int32, (E, E), 0)
    em2 = lax.broadcasted_iota(jnp.int32, (E, E), 1)
    lower = (em1 < em2).astype(jnp.float32)
    off = jnp.dot(pc, lower, preferred_element_type=jnp.float32)
    rank0 = jnp.sum(oh0 * csum[:T], axis=1, keepdims=True) - 1.0
    rank1 = jnp.sum(oh1 * csum[T:], axis=1, keepdims=True) - 1.0
    off0 = jnp.sum(oh0 * off, axis=1, keepdims=True)
    off1 = jnp.sum(oh1 * off, axis=1, keepdims=True)
    slot_ref[...] = jnp.concatenate(
        [off0 + rank0, off1 + rank1], axis=1).astype(jnp.int32)
    ends = off + pc
    bio = (lax.broadcasted_iota(jnp.int32, (NBLK, E), 0) * BM).astype(jnp.float32)
    bef = jnp.sum((bio >= ends).astype(jnp.int32), axis=1, keepdims=True)
    be_ref[...] = jnp.minimum(bef, E - 1)
    tp_ref[...] = jnp.sum(pc).astype(jnp.int32).reshape(1, 1)


def _meta(sel2):
    return pl.pallas_call(
        _meta_body,
        grid=(1,),
        in_specs=[pl.BlockSpec((T, 2), lambda i: (0, 0))],
        out_specs=[
            pl.BlockSpec((T, 2), lambda i: (0, 0)),
            pl.BlockSpec((NBLK, 1), lambda i: (0, 0)),
            pl.BlockSpec((1, 1), lambda i: (0, 0)),
        ],
        out_shape=[
            jax.ShapeDtypeStruct((T, 2), jnp.int32),
            jax.ShapeDtypeStruct((NBLK, 1), jnp.int32),
            jax.ShapeDtypeStruct((1, 1), jnp.int32),
        ],
    )(sel2)


def _egemm_body(be_ref, tp_ref, xg_ref, w1_ref, w3_ref, w2_ref, yg_ref):
    i = pl.program_id(0)

    @pl.when(i * BM < tp_ref[0])
    def _():
        xb = xg_ref[...]
        h1 = jnp.dot(xb, w1_ref[0], preferred_element_type=jnp.float32)
        h1 = h1 * (1.0 / (1.0 + jnp.exp(-h1)))
        h3 = jnp.dot(xb, w3_ref[0], preferred_element_type=jnp.float32)
        yg_ref[...] = jnp.dot(h1 * h3, w2_ref[0],
                              preferred_element_type=jnp.float32)


def _egemm(be, tp, xg, w1, w3, w2):
    grid_spec = pltpu.PrefetchScalarGridSpec(
        num_scalar_prefetch=2,
        grid=(NBLK,),
        in_specs=[
            pl.BlockSpec((BM, D), lambda i, be, tp: (i, 0)),
            pl.BlockSpec((1, D, HID), lambda i, be, tp: (be[i], 0, 0)),
            pl.BlockSpec((1, D, HID), lambda i, be, tp: (be[i], 0, 0)),
            pl.BlockSpec((1, HID, D), lambda i, be, tp: (be[i], 0, 0)),
        ],
        out_specs=pl.BlockSpec((BM, D), lambda i, be, tp: (i, 0)),
    )
    return pl.pallas_call(
        _egemm_body,
        grid_spec=grid_spec,
        out_shape=jax.ShapeDtypeStruct((PAD, D), jnp.float32),
    )(be, tp, xg, w1, w3, w2)


def _combine_body(x_ref, w_ref, sel_ref, y0_ref, y1_ref, o_ref):
    w = w_ref[...]
    sel = sel_ref[...]
    lo = sel[:, 0:1] < sel[:, 1:2]
    wlo = jnp.where(lo, w[:, 0:1], w[:, 1:2])
    whi = jnp.where(lo, w[:, 1:2], w[:, 0:1])
    y0 = y0_ref[...]
    y1 = y1_ref[...]
    ylo = jnp.where(lo, y0, y1)
    yhi = jnp.where(lo, y1, y0)
    o_ref[...] = x_ref[...] + (wlo * ylo + whi * yhi)


def _combine(x, wt2, sel2, y0, y1):
    n = T // RB
    return pl.pallas_call(
        _combine_body,
        grid=(n,),
        in_specs=[
            pl.BlockSpec((RB, D), lambda i: (i, 0)),
            pl.BlockSpec((RB, 2), lambda i: (i, 0)),
            pl.BlockSpec((RB, 2), lambda i: (i, 0)),
            pl.BlockSpec((RB, D), lambda i: (i, 0)),
            pl.BlockSpec((RB, D), lambda i: (i, 0)),
        ],
        out_specs=pl.BlockSpec((RB, D), lambda i: (i, 0)),
        out_shape=jax.ShapeDtypeStruct((T, D), jnp.float32),
    )(x, wt2, sel2, y0, y1)


def _moe_routed(x, xf, sel2, wt2, w1, w3, w2):
    slot2, be, tp = _meta(sel2)
    xg = _dispatch(xf, slot2[:, 0], slot2[:, 1])
    yg = _egemm(be.reshape(NBLK), tp.reshape(1), xg, w1, w3, w2)
    y0 = _gather_rows(yg, slot2[:, 0])
    y1 = _gather_rows(yg, slot2[:, 1])
    return _combine(x, wt2, sel2, y0, y1)


def _rms_body(x_ref, ln_ref, o_ref):
    x = x_ref[...]
    ms = jnp.mean(x * x, axis=-1, keepdims=True)
    o_ref[...] = x * lax.rsqrt(ms + 1e-5) * ln_ref[...]


def _rmsnorm(x, ln):
    n = T // RB
    return pl.pallas_call(
        _rms_body,
        grid=(n,),
        in_specs=[
            pl.BlockSpec((RB, D), lambda i: (i, 0)),
            pl.BlockSpec((1, D), lambda i: (0, 0)),
        ],
        out_specs=pl.BlockSpec((RB, D), lambda i: (i, 0)),
        out_shape=jax.ShapeDtypeStruct((T, D), jnp.float32),
    )(x, ln.reshape(1, D))


def _head_body(xn_ref, wt_ref, tgt_ref, logits_ref, loss_ref,
               m_ref, s_ref, tl_ref):
    v = pl.program_id(0)
    lg = lax.dot_general(xn_ref[...], wt_ref[...], (((1,), (1,)), ((), ())),
                         preferred_element_type=jnp.float32)
    logits_ref[...] = lg
    col = lax.broadcasted_iota(jnp.int32, (T, VB), 1) + v * VB
    hit = col == tgt_ref[...]
    tl_part = jnp.sum(jnp.where(hit, lg, 0.0), axis=1, keepdims=True)
    rm = jnp.max(lg, axis=1, keepdims=True)

    @pl.when(v == 0)
    def _():
        m_ref[...] = jnp.full((T, 1), -1e30, jnp.float32)
        s_ref[...] = jnp.zeros((T, 1), jnp.float32)
        tl_ref[...] = jnp.zeros((T, 1), jnp.float32)

    m_old = m_ref[...]
    m_new = jnp.maximum(m_old, rm)
    s_ref[...] = s_ref[...] * jnp.exp(m_old - m_new) + jnp.sum(
        jnp.exp(lg - m_new), axis=1, keepdims=True)
    m_ref[...] = m_new
    tl_ref[...] += tl_part

    @pl.when(v == NV - 1)
    def _():
        loss_ref[...] = jnp.mean(
            m_ref[...] + jnp.log(s_ref[...]) - tl_ref[...]).reshape(1, 1)


def _head(xn, wte, tgt):
    return pl.pallas_call(
        _head_body,
        grid=(NV,),
        in_specs=[
            pl.BlockSpec((T, D), lambda v: (0, 0)),
            pl.BlockSpec((VB, D), lambda v: (v, 0)),
            pl.BlockSpec((T, 1), lambda v: (0, 0)),
        ],
        out_specs=[
            pl.BlockSpec((T, VB), lambda v: (0, v)),
            pl.BlockSpec((1, 1), lambda v: (0, 0)),
        ],
        out_shape=[
            jax.ShapeDtypeStruct((T, V), jnp.float32),
            jax.ShapeDtypeStruct((1, 1), jnp.float32),
        ],
        scratch_shapes=[
            pltpu.VMEM((T, 1), jnp.float32),
            pltpu.VMEM((T, 1), jnp.float32),
            pltpu.VMEM((T, 1), jnp.float32),
        ],
    )(xn, wte, tgt)


def _moe_dense_jax(h, lp):
    b, t, c = h.shape
    xf = h.reshape(-1, c)
    gl = xf @ lp['gate']
    w, sel = jax.lax.top_k(gl, 2)
    w = jax.nn.softmax(w.astype(jnp.float32), axis=-1).astype(xf.dtype)
    out = jnp.zeros_like(xf)
    for j in range(E):
        mw = jnp.sum(w * (sel == j).astype(w.dtype), axis=-1)
        eo = (jax.nn.silu(xf @ lp['w1'][j]) * (xf @ lp['w3'][j])) @ lp['w2'][j]
        out = out + mw[:, None] * eo
    return out.reshape(b, t, c)


def kernel(params, idx, targets):
    wte = params['wte']
    x = _embed(wte, idx.reshape(T)).reshape(1, T, D)
    nl = len(params['layers'])
    for li, lp in enumerate(params['layers']):
        x = x + _attention(_norm(x, lp['ln1']), lp)
        xn2 = _norm(x, lp['ln2'])
        if li < nl - 1:
            x = x + _moe_dense_jax(xn2, lp)
        else:
            xf = xn2.reshape(-1, D)
            gl = xf @ lp['gate']
            w, sel = jax.lax.top_k(gl, 2)
            w = jax.nn.softmax(w.astype(jnp.float32), axis=-1).astype(xf.dtype)
            x = _moe_routed(x.reshape(T, D), xf, sel, w,
                            lp['w1'], lp['w3'], lp['w2']).reshape(1, T, D)
    xn = _rmsnorm(x.reshape(T, D), params['ln_f'])
    logits, loss = _head(xn, wte, targets.reshape(T, 1))
    return logits.reshape(1, T, V), loss.reshape(())


# speedup vs baseline: 1.3632x; 1.0021x over previous
import functools

import jax
import jax.numpy as jnp
import numpy as np
from jax import lax
from jax.experimental import pallas as pl
from jax.experimental.pallas import tpu as pltpu
from jax.experimental.pallas import tpu_sc as plsc

V = 32000
D = 768
NH = 12
HD = D // NH
E = 8
HID = int(2 * D / 3) * 4
T = 2048
THETA = 10000.0

_freqs = 1.0 / THETA ** (np.arange(0, HD, 2)[: HD // 2].astype(np.float32) / HD)
_ang = np.outer(np.arange(T * 2).astype(np.float32), _freqs)
_COS = jnp.asarray(np.cos(_ang), dtype=jnp.float32)
_SIN = jnp.asarray(np.sin(_ang), dtype=jnp.float32)

RB = 256
VB = 1280
NV = V // VB
BM = 128
PAD = 2 * T + E * BM
NBLK = PAD // BM


def _norm(x, w):
    xf = x.astype(jnp.float32)
    out = xf * lax.rsqrt(jnp.mean(xf * xf, axis=-1, keepdims=True) + 1e-5)
    return out.astype(x.dtype) * w


def _rotary(x):
    t = x.shape[1]
    xr = x.astype(jnp.float32).reshape(x.shape[0], x.shape[1], x.shape[2], -1, 2)
    x0 = xr[..., 0]
    x1 = xr[..., 1]
    c = _COS[:t].reshape(1, t, 1, -1)
    s = _SIN[:t].reshape(1, t, 1, -1)
    o0 = x0 * c - x1 * s
    o1 = x0 * s + x1 * c
    return jnp.stack([o0, o1], axis=-1).reshape(x.shape).astype(x.dtype)


def _attention(h, lp):
    b, t, c = h.shape
    xq = (h @ lp['wq']).reshape(b, t, NH, HD)
    xk = (h @ lp['wk']).reshape(b, t, NH, HD)
    xv = (h @ lp['wv']).reshape(b, t, NH, HD)
    xq = _rotary(xq)
    xk = _rotary(xk)
    q = xq.transpose(0, 2, 1, 3)
    k = xk.transpose(0, 2, 1, 3)
    v = xv.transpose(0, 2, 1, 3)
    scores = (q @ k.transpose(0, 1, 3, 2)) / jnp.sqrt(jnp.float32(HD))
    mask = jnp.tril(jnp.ones((t, t), dtype=bool))
    scores = jnp.where(mask[None, None], scores, jnp.float32(-1e9))
    p = jax.nn.softmax(scores, axis=-1)
    y = (p @ v).transpose(0, 2, 1, 3).reshape(b, t, c)
    return y @ lp['wo']


def _gather_rows(table, idx):
    info = plsc.get_sparse_core_info()
    nc, ns = info.num_cores, info.num_subcores
    bpw = T // (nc * ns)
    mesh = plsc.VectorSubcoreMesh(core_axis_name="c", subcore_axis_name="s")

    @functools.partial(
        pl.kernel,
        mesh=mesh,
        out_type=jax.ShapeDtypeStruct((T, D), jnp.float32),
        scratch_types=[
            pltpu.VMEM((bpw,), jnp.int32),
            pltpu.VMEM((bpw, D), jnp.float32),
            pltpu.SemaphoreType.DMA,
        ],
    )
    def k(table_hbm, idx_hbm, out_hbm, idx_v, rows_v, sem):
        wid = lax.axis_index("s") * nc + lax.axis_index("c")
        base = wid * bpw
        pltpu.sync_copy(idx_hbm.at[pl.ds(base, bpw)], idx_v)
        pltpu.async_copy(table_hbm.at[idx_v], rows_v, sem).wait()
        pltpu.sync_copy(rows_v, out_hbm.at[pl.ds(base, bpw)])

    return k(table, idx)


def _embed(table, idx):
    return _gather_rows(table, idx)


def _dispatch(xn, slk0, slk1):
    info = plsc.get_sparse_core_info()
    nc, ns = info.num_cores, info.num_subcores
    bpw = T // (nc * ns)
    mesh = plsc.VectorSubcoreMesh(core_axis_name="c", subcore_axis_name="s")

    @functools.partial(
        pl.kernel,
        mesh=mesh,
        out_type=jax.ShapeDtypeStruct((PAD, D), jnp.float32),
        scratch_types=[
            pltpu.VMEM((bpw,), jnp.int32),
            pltpu.VMEM((bpw,), jnp.int32),
            pltpu.VMEM((bpw, D), jnp.float32),
            pltpu.SemaphoreType.DMA,
            pltpu.SemaphoreType.DMA,
        ],
    )
    def k(xn_hbm, s0_hbm, s1_hbm, out_hbm, i0_v, i1_v, rows_v, sem0, sem1):
        wid = lax.axis_index("s") * nc + lax.axis_index("c")
        base = wid * bpw
        pltpu.sync_copy(xn_hbm.at[pl.ds(base, bpw)], rows_v)
        pltpu.sync_copy(s0_hbm.at[pl.ds(base, bpw)], i0_v)
        pltpu.sync_copy(s1_hbm.at[pl.ds(base, bpw)], i1_v)
        c0 = pltpu.async_copy(rows_v, out_hbm.at[i0_v], sem0)
        c1 = pltpu.async_copy(rows_v, out_hbm.at[i1_v], sem1)
        c0.wait()
        c1.wait()

    return k(xn, slk0, slk1)


def _block_csum(oh):
    tril = (lax.broadcasted_iota(jnp.int32, (128, 128), 0)
            >= lax.broadcasted_iota(jnp.int32, (128, 128), 1)).astype(jnp.float32)
    outs = []
    tot = jnp.zeros((1, E), jnp.float32)
    for c in range(T // 128):
        ch = oh[c * 128:(c + 1) * 128]
        w = jnp.dot(tril, ch, preferred_element_type=jnp.float32)
        outs.append(w + tot)
        tot = tot + jnp.sum(ch, axis=0, keepdims=True)
    return jnp.concatenate(outs, axis=0), tot


def _meta_body(sel_ref, slot_ref, be_ref, tp_ref):
    sel = sel_ref[...]
    s0 = sel[:, 0:1]
    s1 = sel[:, 1:2]
    lanes = lax.broadcasted_iota(jnp.int32, (T, E), 1)
    oh0 = (lanes == s0).astype(jnp.float32)
    oh1 = (lanes == s1).astype(jnp.float32)
    csum0, tot0 = _block_csum(oh0)
    csum1, tot1 = _block_csum(oh1)
    csum1 = csum1 + tot0
    csum = jnp.concatenate([csum0, csum1], axis=0)
    counts = tot0 + tot1
    pc = jnp.ceil(counts * (1.0 / BM)) * BM
    em1 = lax.broadcasted_iota(jnp.int32, (E, E), 0)
    em2 = lax.broadcasted_iota(jnp.int32, (E, E), 1)
    lower = (em1 < em2).astype(jnp.float32)
    off = jnp.dot(pc, lower, preferred_element_type=jnp.float32)
    rank0 = jnp.sum(oh0 * csum[:T], axis=1, keepdims=True) - 1.0
    rank1 = jnp.sum(oh1 * csum[T:], axis=1, keepdims=True) - 1.0
    off0 = jnp.sum(oh0 * off, axis=1, keepdims=True)
    off1 = jnp.sum(oh1 * off, axis=1, keepdims=True)
    slot_ref[...] = jnp.concatenate(
        [off0 + rank0, off1 + rank1], axis=1).astype(jnp.int32)
    ends = off + pc
    bio = (lax.broadcasted_iota(jnp.int32, (NBLK, E), 0) * BM).astype(jnp.float32)
    bef = jnp.sum((bio >= ends).astype(jnp.int32), axis=1, keepdims=True)
    be_ref[...] = jnp.minimum(bef, E - 1)
    tp_ref[...] = jnp.sum(pc).astype(jnp.int32).reshape(1, 1)


def _meta(sel2):
    return pl.pallas_call(
        _meta_body,
        grid=(1,),
        in_specs=[pl.BlockSpec((T, 2), lambda i: (0, 0))],
        out_specs=[
            pl.BlockSpec((T, 2), lambda i: (0, 0)),
            pl.BlockSpec((NBLK, 1), lambda i: (0, 0)),
            pl.BlockSpec((1, 1), lambda i: (0, 0)),
        ],
        out_shape=[
            jax.ShapeDtypeStruct((T, 2), jnp.int32),
            jax.ShapeDtypeStruct((NBLK, 1), jnp.int32),
            jax.ShapeDtypeStruct((1, 1), jnp.int32),
        ],
    )(sel2)


def _egemm_body(be_ref, tp_ref, xg_ref, w1_ref, w3_ref, w2_ref, yg_ref):
    i = pl.program_id(0)

    @pl.when(i * BM < tp_ref[0])
    def _():
        xb = xg_ref[...]
        h1 = jnp.dot(xb, w1_ref[0], preferred_element_type=jnp.float32)
        h1 = h1 * (1.0 / (1.0 + jnp.exp(-h1)))
        h3 = jnp.dot(xb, w3_ref[0], preferred_element_type=jnp.float32)
        yg_ref[...] = jnp.dot(h1 * h3, w2_ref[0],
                              preferred_element_type=jnp.float32)


def _egemm(be, tp, xg, w1, w3, w2):
    grid_spec = pltpu.PrefetchScalarGridSpec(
        num_scalar_prefetch=2,
        grid=(NBLK,),
        in_specs=[
            pl.BlockSpec((BM, D), lambda i, be, tp: (i, 0)),
            pl.BlockSpec((1, D, HID), lambda i, be, tp: (be[i], 0, 0)),
            pl.BlockSpec((1, D, HID), lambda i, be, tp: (be[i], 0, 0)),
            pl.BlockSpec((1, HID, D), lambda i, be, tp: (be[i], 0, 0)),
        ],
        out_specs=pl.BlockSpec((BM, D), lambda i, be, tp: (i, 0)),
    )
    return pl.pallas_call(
        _egemm_body,
        grid_spec=grid_spec,
        out_shape=jax.ShapeDtypeStruct((PAD, D), jnp.float32),
    )(be, tp, xg, w1, w3, w2)


def _combine_body(x_ref, w_ref, sel_ref, y0_ref, y1_ref, o_ref):
    w = w_ref[...]
    sel = sel_ref[...]
    lo = sel[:, 0:1] < sel[:, 1:2]
    wlo = jnp.where(lo, w[:, 0:1], w[:, 1:2])
    whi = jnp.where(lo, w[:, 1:2], w[:, 0:1])
    y0 = y0_ref[...]
    y1 = y1_ref[...]
    ylo = jnp.where(lo, y0, y1)
    yhi = jnp.where(lo, y1, y0)
    o_ref[...] = x_ref[...] + (wlo * ylo + whi * yhi)


def _combine(x, wt2, sel2, y0, y1):
    n = T // RB
    return pl.pallas_call(
        _combine_body,
        grid=(n,),
        in_specs=[
            pl.BlockSpec((RB, D), lambda i: (i, 0)),
            pl.BlockSpec((RB, 2), lambda i: (i, 0)),
            pl.BlockSpec((RB, 2), lambda i: (i, 0)),
            pl.BlockSpec((RB, D), lambda i: (i, 0)),
            pl.BlockSpec((RB, D), lambda i: (i, 0)),
        ],
        out_specs=pl.BlockSpec((RB, D), lambda i: (i, 0)),
        out_shape=jax.ShapeDtypeStruct((T, D), jnp.float32),
    )(x, wt2, sel2, y0, y1)


def _moe_routed(x, xf, sel2, wt2, w1, w3, w2):
    slot2, be, tp = _meta(sel2)
    xg = _dispatch(xf, slot2[:, 0], slot2[:, 1])
    yg = _egemm(be.reshape(NBLK), tp.reshape(1), xg, w1, w3, w2)
    y0 = _gather_rows(yg, slot2[:, 0])
    y1 = _gather_rows(yg, slot2[:, 1])
    return _combine(x, wt2, sel2, y0, y1)


def _rms_body(x_ref, ln_ref, o_ref):
    x = x_ref[...]
    ms = jnp.mean(x * x, axis=-1, keepdims=True)
    o_ref[...] = x * lax.rsqrt(ms + 1e-5) * ln_ref[...]


def _rmsnorm(x, ln):
    n = T // RB
    return pl.pallas_call(
        _rms_body,
        grid=(n,),
        in_specs=[
            pl.BlockSpec((RB, D), lambda i: (i, 0)),
            pl.BlockSpec((1, D), lambda i: (0, 0)),
        ],
        out_specs=pl.BlockSpec((RB, D), lambda i: (i, 0)),
        out_shape=jax.ShapeDtypeStruct((T, D), jnp.float32),
    )(x, ln.reshape(1, D))


def _head_body(xn_ref, wt_ref, tgt_ref, logits_ref, loss_ref,
               m_ref, s_ref, tl_ref):
    v = pl.program_id(0)
    lg = lax.dot_general(xn_ref[...], wt_ref[...], (((1,), (1,)), ((), ())),
                         preferred_element_type=jnp.float32)
    logits_ref[...] = lg
    col = lax.broadcasted_iota(jnp.int32, (T, VB), 1) + v * VB
    hit = col == tgt_ref[...]
    tl_part = jnp.sum(jnp.where(hit, lg, 0.0), axis=1, keepdims=True)
    rm = jnp.max(lg, axis=1, keepdims=True)

    @pl.when(v == 0)
    def _():
        m_ref[...] = jnp.full((T, 1), -1e30, jnp.float32)
        s_ref[...] = jnp.zeros((T, 1), jnp.float32)
        tl_ref[...] = jnp.zeros((T, 1), jnp.float32)

    m_old = m_ref[...]
    m_new = jnp.maximum(m_old, rm)
    s_ref[...] = s_ref[...] * jnp.exp(m_old - m_new) + jnp.sum(
        jnp.exp(lg - m_new), axis=1, keepdims=True)
    m_ref[...] = m_new
    tl_ref[...] += tl_part

    @pl.when(v == NV - 1)
    def _():
        loss_ref[...] = jnp.mean(
            m_ref[...] + jnp.log(s_ref[...]) - tl_ref[...]).reshape(1, 1)


def _head(xn, wte, tgt):
    return pl.pallas_call(
        _head_body,
        grid=(NV,),
        in_specs=[
            pl.BlockSpec((T, D), lambda v: (0, 0)),
            pl.BlockSpec((VB, D), lambda v: (v, 0)),
            pl.BlockSpec((T, 1), lambda v: (0, 0)),
        ],
        out_specs=[
            pl.BlockSpec((T, VB), lambda v: (0, v)),
            pl.BlockSpec((1, 1), lambda v: (0, 0)),
        ],
        out_shape=[
            jax.ShapeDtypeStruct((T, V), jnp.float32),
            jax.ShapeDtypeStruct((1, 1), jnp.float32),
        ],
        scratch_shapes=[
            pltpu.VMEM((T, 1), jnp.float32),
            pltpu.VMEM((T, 1), jnp.float32),
            pltpu.VMEM((T, 1), jnp.float32),
        ],
    )(xn, wte, tgt)


def _moe_dense_jax(h, lp):
    b, t, c = h.shape
    xf = h.reshape(-1, c)
    gl = xf @ lp['gate']
    w, sel = jax.lax.top_k(gl, 2)
    w = jax.nn.softmax(w.astype(jnp.float32), axis=-1).astype(xf.dtype)
    out = jnp.zeros_like(xf)
    for j in range(E):
        mw = jnp.sum(w * (sel == j).astype(w.dtype), axis=-1)
        eo = (jax.nn.silu(xf @ lp['w1'][j]) * (xf @ lp['w3'][j])) @ lp['w2'][j]
        out = out + mw[:, None] * eo
    return out.reshape(b, t, c)


def kernel(params, idx, targets):
    wte = params['wte']
    x = _embed(wte, idx.reshape(T)).reshape(1, T, D)
    x = lax.optimization_barrier(x)
    nl = len(params['layers'])
    for li, lp in enumerate(params['layers']):
        x = x + _attention(_norm(x, lp['ln1']), lp)
        xn2 = _norm(x, lp['ln2'])
        if li < nl - 1:
            x = lax.optimization_barrier(x + _moe_dense_jax(xn2, lp))
        else:
            xf = xn2.reshape(-1, D)
            gl = xf @ lp['gate']
            w, sel = jax.lax.top_k(gl, 2)
            w = jax.nn.softmax(w.astype(jnp.float32), axis=-1).astype(xf.dtype)
            x = _moe_routed(x.reshape(T, D), xf, sel, w,
                            lp['w1'], lp['w3'], lp['w2']).reshape(1, T, D)
    xn = _rmsnorm(x.reshape(T, D), params['ln_f'])
    logits, loss = _head(xn, wte, targets.reshape(T, 1))
    return logits.reshape(1, T, V), loss.reshape(())
